# baseline (speedup 1.0000x reference)
import functools

import jax
import jax.numpy as jnp
from jax import lax
from jax.experimental import pallas as pl
from jax.experimental.pallas import tpu as pltpu
from jax.experimental.pallas import tpu_sc as plsc

_IDX_CHUNK = 128


@functools.lru_cache(maxsize=None)
def _make_sc_gather(B, V, D):
    info = plsc.get_sparse_core_info()
    num_workers = info.num_cores * info.num_subcores
    b_per_w = B // num_workers
    n_chunks = b_per_w // _IDX_CHUNK
    mesh = plsc.VectorSubcoreMesh(core_axis_name="c", subcore_axis_name="s")

    @functools.partial(
        pl.kernel,
        mesh=mesh,
        out_type=jax.ShapeDtypeStruct((B, D), jnp.float32),
        compiler_params=pltpu.CompilerParams(use_tc_tiling_on_sc=False),
        scratch_types=[
            pltpu.VMEM((n_chunks, _IDX_CHUNK), jnp.int32),
            pltpu.VMEM((b_per_w, D), jnp.float32),
            pltpu.SemaphoreType.DMA,
        ],
    )
    def gather_kernel(table_hbm, idx_hbm, out_hbm, idx_v, rows_v, sem):
        wid = lax.axis_index("s") * info.num_cores + lax.axis_index("c")
        pltpu.sync_copy(idx_hbm.at[pl.ds(wid * n_chunks, n_chunks)], idx_v)
        copies = [
            pltpu.async_copy(
                table_hbm.at[idx_v.at[j]],
                rows_v.at[pl.ds(j * _IDX_CHUNK, _IDX_CHUNK)],
                sem,
            )
            for j in range(n_chunks)
        ]
        for c in copies:
            c.wait()
        pltpu.sync_copy(rows_v, out_hbm.at[pl.ds(wid * b_per_w, b_per_w)])

    def run(table, x):
        x2d = x.reshape(num_workers * n_chunks, _IDX_CHUNK)
        return gather_kernel(table, x2d)

    return run


def _tc_linear_body(h_ref, wm_ref, bm_ref, wl_ref, bl_ref, mu_ref, lv_ref):
    hb = h_ref[...]
    mu_ref[...] = (
        jnp.dot(hb, wm_ref[...], preferred_element_type=jnp.float32) + bm_ref[...]
    )
    lv_ref[...] = (
        jnp.dot(hb, wl_ref[...], preferred_element_type=jnp.float32) + bl_ref[...]
    )


@functools.lru_cache(maxsize=None)
def _make_tc_linear(B, D, O, grid):
    blk = B // grid
    return pl.pallas_call(
        _tc_linear_body,
        grid=(grid,),
        in_specs=[
            pl.BlockSpec((blk, D), lambda i: (i, 0)),
            pl.BlockSpec((D, O), lambda i: (0, 0)),
            pl.BlockSpec((1, O), lambda i: (0, 0)),
            pl.BlockSpec((D, O), lambda i: (0, 0)),
            pl.BlockSpec((1, O), lambda i: (0, 0)),
        ],
        out_specs=[
            pl.BlockSpec((blk, O), lambda i: (i, 0)),
            pl.BlockSpec((blk, O), lambda i: (i, 0)),
        ],
        out_shape=[
            jax.ShapeDtypeStruct((B, O), jnp.float32),
            jax.ShapeDtypeStruct((B, O), jnp.float32),
        ],
    )


def kernel(x, table, W_mu, b_mu, W_logvar, b_logvar):
    B = x.shape[0]
    V, D = table.shape
    O = W_mu.shape[1]
    h = _make_sc_gather(B, V, D)(table, x)
    mu, logvar = _make_tc_linear(B, D, O, 8)(
        h, W_mu, b_mu.reshape(1, O), W_logvar, b_logvar.reshape(1, O)
    )
    return (mu, logvar)


# speedup vs baseline: 1.0008x; 1.0008x over previous
import functools

import jax
import jax.numpy as jnp
from jax import lax
from jax.experimental import pallas as pl
from jax.experimental.pallas import tpu as pltpu
from jax.experimental.pallas import tpu_sc as plsc

_IDX_CHUNK = 128


@functools.lru_cache(maxsize=None)
def _make_sc_gather(B, V, D):
    info = plsc.get_sparse_core_info()
    num_workers = info.num_cores * info.num_subcores
    b_per_w = B // num_workers
    n_chunks = b_per_w // _IDX_CHUNK
    mesh = plsc.VectorSubcoreMesh(core_axis_name="c", subcore_axis_name="s")

    @functools.partial(
        pl.kernel,
        mesh=mesh,
        out_type=jax.ShapeDtypeStruct((B, D), jnp.float32),
        compiler_params=pltpu.CompilerParams(use_tc_tiling_on_sc=False),
        scratch_types=[
            pltpu.VMEM((n_chunks, _IDX_CHUNK), jnp.int32),
            pltpu.VMEM((b_per_w, D), jnp.float32),
            pltpu.SemaphoreType.DMA,
        ],
    )
    def gather_kernel(table_hbm, idx_hbm, out_hbm, idx_v, rows_v, sem):
        wid = lax.axis_index("s") * info.num_cores + lax.axis_index("c")
        pltpu.sync_copy(idx_hbm.at[pl.ds(wid * n_chunks, n_chunks)], idx_v)
        copies = [
            pltpu.async_copy(
                table_hbm.at[idx_v.at[j]],
                rows_v.at[pl.ds(j * _IDX_CHUNK, _IDX_CHUNK)],
                sem,
            )
            for j in range(n_chunks)
        ]
        for c in copies:
            c.wait()
        pltpu.sync_copy(rows_v, out_hbm.at[pl.ds(wid * b_per_w, b_per_w)])

    def run(table, x):
        x2d = x.reshape(num_workers * n_chunks, _IDX_CHUNK)
        table_lin = table.reshape(-1).reshape(V, D)
        return gather_kernel(table_lin, x2d)

    return run


def _tc_linear_body(h_ref, wm_ref, bm_ref, wl_ref, bl_ref, mu_ref, lv_ref):
    hb = h_ref[...]
    mu_ref[...] = (
        jnp.dot(hb, wm_ref[...], preferred_element_type=jnp.float32) + bm_ref[...]
    )
    lv_ref[...] = (
        jnp.dot(hb, wl_ref[...], preferred_element_type=jnp.float32) + bl_ref[...]
    )


@functools.lru_cache(maxsize=None)
def _make_tc_linear(B, D, O, grid):
    blk = B // grid
    return pl.pallas_call(
        _tc_linear_body,
        grid=(grid,),
        in_specs=[
            pl.BlockSpec((blk, D), lambda i: (i, 0)),
            pl.BlockSpec((D, O), lambda i: (0, 0)),
            pl.BlockSpec((1, O), lambda i: (0, 0)),
            pl.BlockSpec((D, O), lambda i: (0, 0)),
            pl.BlockSpec((1, O), lambda i: (0, 0)),
        ],
        out_specs=[
            pl.BlockSpec((blk, O), lambda i: (i, 0)),
            pl.BlockSpec((blk, O), lambda i: (i, 0)),
        ],
        out_shape=[
            jax.ShapeDtypeStruct((B, O), jnp.float32),
            jax.ShapeDtypeStruct((B, O), jnp.float32),
        ],
    )


def kernel(x, table, W_mu, b_mu, W_logvar, b_logvar):
    B = x.shape[0]
    V, D = table.shape
    O = W_mu.shape[1]
    h = _make_sc_gather(B, V, D)(table, x)
    mu, logvar = _make_tc_linear(B, D, O, 8)(
        h, W_mu, b_mu.reshape(1, O), W_logvar, b_logvar.reshape(1, O)
    )
    return (mu, logvar)
